# Initial kernel scaffold; baseline (speedup 1.0000x reference)
import jax
import jax.numpy as jnp
from jax.experimental import pallas as pl


def kernel(x, timesteps, graph_rows, graph_cols, graph_vals, W_emb, b_emb, W1, b1, W2, b2):
    raise NotImplementedError("write your pallas kernel here")


# speedup vs baseline: 5.4134x; 5.4134x over previous
import functools
import math

import jax
import jax.numpy as jnp
from jax import lax
from jax.experimental import pallas as pl
from jax.experimental.pallas import tpu as pltpu
from jax.experimental.pallas import tpu_sc as plsc

N = 4096
B = 1024
EMB = 128
HID = 1024

NC = 2
NS = 16
SC_ROWS = 256
N_PASSES = N // (NC * SC_ROWS)
ROWS_PER_TILE = SC_ROWS // NS
NCHUNK = 8


def _densify_body(rows_hbm, cols_hbm, vals_hbm, zeros_hbm, d_hbm,
                  rows_v, cols_v, vals_v, idx_buf, val_buf, shared, sem):
    chunk = rows_v.shape[0]
    share = chunk * NCHUNK
    cid = lax.axis_index("c")
    sid = lax.axis_index("s")

    for p in range(N_PASSES):
        base_c = (cid * N_PASSES + p) * SC_ROWS

        pltpu.sync_copy(zeros_hbm.at[pl.ds(sid * ROWS_PER_TILE * N,
                                           ROWS_PER_TILE * N)],
                        shared.at[pl.ds(sid * ROWS_PER_TILE * N,
                                        ROWS_PER_TILE * N)])
        plsc.subcore_barrier()

        for c in range(NCHUNK):
            off = sid * share + c * chunk
            pltpu.sync_copy(rows_hbm.at[pl.ds(off, chunk)], rows_v)
            pltpu.sync_copy(cols_hbm.at[pl.ds(off, chunk)], cols_v)
            pltpu.sync_copy(vals_hbm.at[pl.ds(off, chunk)], vals_v)

            def scan_body(i, _):
                r16 = rows_v[pl.ds(i * 16, 16)]
                c16 = cols_v[pl.ds(i * 16, 16)]
                v16 = vals_v[pl.ds(i * 16, 16)]
                lc = c16 - base_c
                m = (lc >= 0) & (lc < SC_ROWS)
                idx_buf[pl.ds(i * 16, 16)] = jnp.where(m, lc * N + r16, -1)
                val_buf[pl.ds(i * 16, 16)] = jnp.where(m, v16, 0.0)
                return 0

            lax.fori_loop(0, chunk // 16, scan_body, 0)
            pltpu.sync_copy(val_buf,
                            shared.at[plsc.Indices(idx_buf, ignored_value=-1)],
                            add=True)
        plsc.subcore_barrier()

        src = sid * ROWS_PER_TILE * N
        dst = (base_c + sid * ROWS_PER_TILE) * N
        pltpu.sync_copy(shared.at[pl.ds(src, ROWS_PER_TILE * N)],
                        d_hbm.at[pl.ds(dst, ROWS_PER_TILE * N)])
        plsc.subcore_barrier()


def _densify(rows, cols, vals):
    nnz = rows.shape[0]
    chunk = -(-nnz // (NS * NCHUNK * 16)) * 16
    pad = NS * NCHUNK * chunk - nnz
    rows = jnp.pad(rows, (0, pad))
    cols = jnp.pad(cols, (0, pad), constant_values=-1)
    vals = jnp.pad(vals, (0, pad))
    zeros = jnp.zeros((SC_ROWS * N,), jnp.float32)

    grid_kernel = pl.kernel(
        _densify_body,
        out_type=jax.ShapeDtypeStruct((N * N,), jnp.float32),
        mesh=plsc.VectorSubcoreMesh(core_axis_name="c", subcore_axis_name="s"),
        scratch_types=[
            pltpu.VMEM((chunk,), jnp.int32),
            pltpu.VMEM((chunk,), jnp.int32),
            pltpu.VMEM((chunk,), jnp.float32),
            pltpu.VMEM((chunk,), jnp.int32),
            pltpu.VMEM((chunk,), jnp.float32),
            pltpu.VMEM_SHARED((SC_ROWS * N,), jnp.float32),
            pltpu.SemaphoreType.DMA,
        ],
    )
    return grid_kernel(rows, cols, vals, zeros).reshape(N, N)


_BN = 512


def _graph_layer_body(x_ref, d_ref, out_ref, *, add_x2):
    j = pl.program_id(0)
    xg = x_ref[...]
    acc = lax.dot_general(xg, d_ref[...], (((1,), (0,)), ((), ())),
                          preferred_element_type=jnp.float32,
                          precision=lax.Precision.HIGHEST)
    out_ref[...] = acc + x_ref[:, pl.ds(j * _BN, _BN)]


def _graph_layer_body_res(x_ref, d_ref, x0_ref, out_ref):
    j = pl.program_id(0)
    xg = x_ref[...]
    acc = lax.dot_general(xg, d_ref[...], (((1,), (0,)), ((), ())),
                          preferred_element_type=jnp.float32,
                          precision=lax.Precision.HIGHEST)
    out_ref[...] = acc + x_ref[:, pl.ds(j * _BN, _BN)] + x0_ref[...]


def _graph_layer(xg, d, x0=None):
    grid = N // _BN
    in_specs = [
        pl.BlockSpec((B, N), lambda j: (0, 0)),
        pl.BlockSpec((N, _BN), lambda j: (0, j)),
    ]
    args = [xg, d]
    if x0 is None:
        body = functools.partial(_graph_layer_body, add_x2=False)
    else:
        body = _graph_layer_body_res
        in_specs.append(pl.BlockSpec((B, _BN), lambda j: (0, j)))
        args.append(x0)
    return pl.pallas_call(
        body,
        grid=(grid,),
        in_specs=in_specs,
        out_specs=pl.BlockSpec((B, _BN), lambda j: (0, j)),
        out_shape=jax.ShapeDtypeStruct((B, N), jnp.float32),
    )(*args)


_BM = 128


def _mlp_body(xx_ref, ts_ref, wemb_ref, bemb_ref, w1_ref, b1_ref, w2_ref, b2_ref,
              out_ref):
    half = EMB // 2
    ts = ts_ref[0, 0, :]
    k = lax.broadcasted_iota(jnp.int32, (1, half), 1).astype(jnp.float32)
    freqs = jnp.exp(k * (-math.log(10000.0) / half))
    args = ts[:, None] * freqs
    te = jnp.concatenate([jnp.cos(args), jnp.sin(args)], axis=-1)
    emb = lax.dot_general(te, wemb_ref[...], (((1,), (1,)), ((), ())),
                          preferred_element_type=jnp.float32,
                          precision=lax.Precision.HIGHEST) + bemb_ref[...]
    xx = xx_ref[...]
    w1 = w1_ref[...]
    pre1 = lax.dot_general(xx, w1[:, :N], (((1,), (1,)), ((), ())),
                           preferred_element_type=jnp.float32,
                           precision=lax.Precision.HIGHEST)
    pre1 = pre1 + lax.dot_general(emb, w1[:, N:], (((1,), (1,)), ((), ())),
                                  preferred_element_type=jnp.float32,
                                  precision=lax.Precision.HIGHEST)
    h1 = jnp.tanh(pre1 + b1_ref[...])
    pre2 = lax.dot_general(h1, w2_ref[...], (((1,), (1,)), ((), ())),
                           preferred_element_type=jnp.float32,
                           precision=lax.Precision.HIGHEST)
    out_ref[...] = jnp.tanh(pre2 + b2_ref[...]) + xx


def _mlp(xx, timesteps, W_emb, b_emb, W1, b1, W2, b2):
    grid = B // _BM
    ts3 = timesteps.astype(jnp.float32).reshape(grid, 1, _BM)
    return pl.pallas_call(
        _mlp_body,
        grid=(grid,),
        in_specs=[
            pl.BlockSpec((_BM, N), lambda i: (i, 0)),
            pl.BlockSpec((1, 1, _BM), lambda i: (i, 0, 0)),
            pl.BlockSpec((EMB, EMB), lambda i: (0, 0)),
            pl.BlockSpec((1, EMB), lambda i: (0, 0)),
            pl.BlockSpec((HID, N + EMB), lambda i: (0, 0)),
            pl.BlockSpec((1, HID), lambda i: (0, 0)),
            pl.BlockSpec((N, HID), lambda i: (0, 0)),
            pl.BlockSpec((1, N), lambda i: (0, 0)),
        ],
        out_specs=pl.BlockSpec((_BM, N), lambda i: (i, 0)),
        out_shape=jax.ShapeDtypeStruct((B, N), jnp.float32),
    )(xx, ts3, W_emb, b_emb.reshape(1, EMB), W1, b1.reshape(1, HID), W2,
      b2.reshape(1, N))


def kernel(x, timesteps, graph_rows, graph_cols, graph_vals,
           W_emb, b_emb, W1, b1, W2, b2):
    rows = graph_rows.astype(jnp.int32)
    cols = graph_cols.astype(jnp.int32)
    vals = graph_vals.astype(jnp.float32)
    d = _densify(rows, cols, vals)
    xg1 = _graph_layer(x, d)
    xx = _graph_layer(xg1, d, x0=x)
    return _mlp(xx, timesteps, W_emb, b_emb, W1, b1, W2, b2)


# speedup vs baseline: 11.2668x; 2.0813x over previous
import functools
import math

import jax
import jax.numpy as jnp
from jax import lax
from jax.experimental import pallas as pl
from jax.experimental.pallas import tpu as pltpu
from jax.experimental.pallas import tpu_sc as plsc

N = 4096
B = 1024
EMB = 128
HID = 1024

NC = 2
NS = 16
SC_ROWS = 256
N_PASSES = N // (NC * SC_ROWS)
ROWS_PER_TILE = SC_ROWS // NS
NCHUNK = 8


def _densify_body(rows_hbm, cols_hbm, vals_hbm, zeros_hbm, d_hbm,
                  rows_v, cols_v, vals_v, idx_buf, val_buf, shared, sem):
    chunk = rows_v.shape[0]
    share = chunk * NCHUNK
    cid = lax.axis_index("c")
    sid = lax.axis_index("s")

    for p in range(N_PASSES):
        base_c = (cid * N_PASSES + p) * SC_ROWS

        pltpu.sync_copy(zeros_hbm.at[pl.ds(sid * ROWS_PER_TILE * N,
                                           ROWS_PER_TILE * N)],
                        shared.at[pl.ds(sid * ROWS_PER_TILE * N,
                                        ROWS_PER_TILE * N)])
        plsc.subcore_barrier()

        for c in range(NCHUNK):
            off = sid * share + c * chunk
            pltpu.sync_copy(rows_hbm.at[pl.ds(off, chunk)], rows_v)
            pltpu.sync_copy(cols_hbm.at[pl.ds(off, chunk)], cols_v)
            pltpu.sync_copy(vals_hbm.at[pl.ds(off, chunk)], vals_v)

            def scan_body(i, _):
                r16 = rows_v[pl.ds(i * 16, 16)]
                c16 = cols_v[pl.ds(i * 16, 16)]
                v16 = vals_v[pl.ds(i * 16, 16)]
                lc = c16 - base_c
                m = (lc >= 0) & (lc < SC_ROWS)
                idx_buf[pl.ds(i * 16, 16)] = jnp.where(m, lc * N + r16, -1)
                val_buf[pl.ds(i * 16, 16)] = jnp.where(m, v16, 0.0)
                return 0

            lax.fori_loop(0, chunk // 16, scan_body, 0)
            pltpu.sync_copy(val_buf,
                            shared.at[plsc.Indices(idx_buf, ignored_value=-1)],
                            add=True)
        plsc.subcore_barrier()

        src = sid * ROWS_PER_TILE * N
        dst = (base_c + sid * ROWS_PER_TILE) * N
        pltpu.sync_copy(shared.at[pl.ds(src, ROWS_PER_TILE * N)],
                        d_hbm.at[pl.ds(dst, ROWS_PER_TILE * N)])
        plsc.subcore_barrier()


def _densify(rows, cols, vals):
    nnz = rows.shape[0]
    chunk = -(-nnz // (NS * NCHUNK * 16)) * 16
    pad = NS * NCHUNK * chunk - nnz
    rows = jnp.pad(rows, (0, pad))
    cols = jnp.pad(cols, (0, pad), constant_values=-1)
    vals = jnp.pad(vals, (0, pad))
    zeros = jnp.zeros((SC_ROWS * N,), jnp.float32)

    grid_kernel = pl.kernel(
        _densify_body,
        out_type=jax.ShapeDtypeStruct((N * N,), jnp.float32),
        mesh=plsc.VectorSubcoreMesh(core_axis_name="c", subcore_axis_name="s"),
        scratch_types=[
            pltpu.VMEM((chunk,), jnp.int32),
            pltpu.VMEM((chunk,), jnp.int32),
            pltpu.VMEM((chunk,), jnp.float32),
            pltpu.VMEM((chunk,), jnp.int32),
            pltpu.VMEM((chunk,), jnp.float32),
            pltpu.VMEM_SHARED((SC_ROWS * N,), jnp.float32),
            pltpu.SemaphoreType.DMA,
        ],
    )
    return grid_kernel(rows, cols, vals, zeros).reshape(N, N)


_BN = 512


def _graph_layer_body(x_ref, d_ref, out_ref, *, add_x2):
    j = pl.program_id(0)
    xg = x_ref[...].astype(jnp.bfloat16)
    acc = lax.dot_general(xg, d_ref[...].astype(jnp.bfloat16),
                          (((1,), (0,)), ((), ())),
                          preferred_element_type=jnp.float32)
    out_ref[...] = acc + x_ref[:, pl.ds(j * _BN, _BN)]


def _graph_layer_body_res(x_ref, d_ref, x0_ref, out_ref):
    j = pl.program_id(0)
    xg = x_ref[...].astype(jnp.bfloat16)
    acc = lax.dot_general(xg, d_ref[...].astype(jnp.bfloat16),
                          (((1,), (0,)), ((), ())),
                          preferred_element_type=jnp.float32)
    out_ref[...] = acc + x_ref[:, pl.ds(j * _BN, _BN)] + x0_ref[...]


def _graph_layer(xg, d, x0=None):
    grid = N // _BN
    in_specs = [
        pl.BlockSpec((B, N), lambda j: (0, 0)),
        pl.BlockSpec((N, _BN), lambda j: (0, j)),
    ]
    args = [xg, d]
    if x0 is None:
        body = functools.partial(_graph_layer_body, add_x2=False)
    else:
        body = _graph_layer_body_res
        in_specs.append(pl.BlockSpec((B, _BN), lambda j: (0, j)))
        args.append(x0)
    return pl.pallas_call(
        body,
        grid=(grid,),
        in_specs=in_specs,
        out_specs=pl.BlockSpec((B, _BN), lambda j: (0, j)),
        out_shape=jax.ShapeDtypeStruct((B, N), jnp.float32),
    )(*args)


_BM = 128


def _mlp_body(xx_ref, ts_ref, wemb_ref, bemb_ref, w1_ref, b1_ref, w2_ref, b2_ref,
              out_ref):
    half = EMB // 2
    ts = ts_ref[0, 0, :]
    k = lax.broadcasted_iota(jnp.int32, (1, half), 1).astype(jnp.float32)
    freqs = jnp.exp(k * (-math.log(10000.0) / half))
    args = ts[:, None] * freqs
    te = jnp.concatenate([jnp.cos(args), jnp.sin(args)], axis=-1)
    emb = lax.dot_general(te, wemb_ref[...], (((1,), (1,)), ((), ())),
                          preferred_element_type=jnp.float32,
                          precision=lax.Precision.HIGHEST) + bemb_ref[...]
    xx = xx_ref[...]
    w1 = w1_ref[...].astype(jnp.bfloat16)
    pre1 = lax.dot_general(xx.astype(jnp.bfloat16), w1[:, :N],
                           (((1,), (1,)), ((), ())),
                           preferred_element_type=jnp.float32)
    pre1 = pre1 + lax.dot_general(emb.astype(jnp.bfloat16), w1[:, N:],
                                  (((1,), (1,)), ((), ())),
                                  preferred_element_type=jnp.float32)
    h1 = jnp.tanh(pre1 + b1_ref[...]).astype(jnp.bfloat16)
    pre2 = lax.dot_general(h1, w2_ref[...].astype(jnp.bfloat16),
                           (((1,), (1,)), ((), ())),
                           preferred_element_type=jnp.float32)
    out_ref[...] = jnp.tanh(pre2 + b2_ref[...]) + xx


def _mlp(xx, timesteps, W_emb, b_emb, W1, b1, W2, b2):
    grid = B // _BM
    ts3 = timesteps.astype(jnp.float32).reshape(grid, 1, _BM)
    return pl.pallas_call(
        _mlp_body,
        grid=(grid,),
        in_specs=[
            pl.BlockSpec((_BM, N), lambda i: (i, 0)),
            pl.BlockSpec((1, 1, _BM), lambda i: (i, 0, 0)),
            pl.BlockSpec((EMB, EMB), lambda i: (0, 0)),
            pl.BlockSpec((1, EMB), lambda i: (0, 0)),
            pl.BlockSpec((HID, N + EMB), lambda i: (0, 0)),
            pl.BlockSpec((1, HID), lambda i: (0, 0)),
            pl.BlockSpec((N, HID), lambda i: (0, 0)),
            pl.BlockSpec((1, N), lambda i: (0, 0)),
        ],
        out_specs=pl.BlockSpec((_BM, N), lambda i: (i, 0)),
        out_shape=jax.ShapeDtypeStruct((B, N), jnp.float32),
    )(xx, ts3, W_emb, b_emb.reshape(1, EMB), W1, b1.reshape(1, HID), W2,
      b2.reshape(1, N))


def kernel(x, timesteps, graph_rows, graph_cols, graph_vals,
           W_emb, b_emb, W1, b1, W2, b2):
    rows = graph_rows.astype(jnp.int32)
    cols = graph_cols.astype(jnp.int32)
    vals = graph_vals.astype(jnp.float32)
    d = _densify(rows, cols, vals)
    xg1 = _graph_layer(x, d)
    xx = _graph_layer(xg1, d, x0=x)
    return _mlp(xx, timesteps, W_emb, b_emb, W1, b1, W2, b2)


# speedup vs baseline: 12.7798x; 1.1343x over previous
import functools
import math

import jax
import jax.numpy as jnp
from jax import lax
from jax.experimental import pallas as pl
from jax.experimental.pallas import tpu as pltpu
from jax.experimental.pallas import tpu_sc as plsc

N = 4096
B = 1024
EMB = 128
HID = 1024

NC = 2
NS = 16
PASS_ROWS = (464, 464, 464, 464, 192)
NCHUNK = 8


def _densify_body(rows_hbm, cols_hbm, vals_hbm, zeros_hbm, d_hbm,
                  rows_v, cols_v, vals_v, idx_buf, val_buf, shared, sem):
    chunk = rows_v.shape[0]
    share = chunk * NCHUNK
    cid = lax.axis_index("c")
    sid = lax.axis_index("s")
    half = N // NC

    pass_base = 0
    for p, rows_p in enumerate(PASS_ROWS):
        base_c = cid * half + pass_base
        rpt = rows_p // NS

        pltpu.sync_copy(zeros_hbm.at[pl.ds(sid * rpt * N, rpt * N)],
                        shared.at[pl.ds(sid * rpt * N, rpt * N)])
        plsc.subcore_barrier()

        for c in range(NCHUNK):
            off = sid * share + c * chunk
            pltpu.sync_copy(rows_hbm.at[pl.ds(off, chunk)], rows_v)
            pltpu.sync_copy(cols_hbm.at[pl.ds(off, chunk)], cols_v)
            pltpu.sync_copy(vals_hbm.at[pl.ds(off, chunk)], vals_v)

            def scan_body(i, _):
                r16 = rows_v[pl.ds(i * 16, 16)]
                c16 = cols_v[pl.ds(i * 16, 16)]
                v16 = vals_v[pl.ds(i * 16, 16)]
                lc = c16 - base_c
                m = (lc >= 0) & (lc < rows_p)
                idx_buf[pl.ds(i * 16, 16)] = jnp.where(m, lc * N + r16, -1)
                val_buf[pl.ds(i * 16, 16)] = jnp.where(m, v16, 0.0)
                return 0

            lax.fori_loop(0, chunk // 16, scan_body, 0)
            pltpu.sync_copy(val_buf,
                            shared.at[plsc.Indices(idx_buf, ignored_value=-1)],
                            add=True)
        plsc.subcore_barrier()

        src = sid * rpt * N
        dst = (base_c + sid * rpt) * N
        pltpu.sync_copy(shared.at[pl.ds(src, rpt * N)],
                        d_hbm.at[pl.ds(dst, rpt * N)])
        plsc.subcore_barrier()
        pass_base += rows_p


def _densify(rows, cols, vals):
    nnz = rows.shape[0]
    chunk = -(-nnz // (NS * NCHUNK * 16)) * 16
    pad = NS * NCHUNK * chunk - nnz
    rows = jnp.pad(rows, (0, pad))
    cols = jnp.pad(cols, (0, pad), constant_values=-1)
    vals = jnp.pad(vals, (0, pad))
    zeros = jnp.zeros((max(PASS_ROWS) * N,), jnp.float32)

    grid_kernel = pl.kernel(
        _densify_body,
        out_type=jax.ShapeDtypeStruct((N * N,), jnp.float32),
        mesh=plsc.VectorSubcoreMesh(core_axis_name="c", subcore_axis_name="s"),
        scratch_types=[
            pltpu.VMEM((chunk,), jnp.int32),
            pltpu.VMEM((chunk,), jnp.int32),
            pltpu.VMEM((chunk,), jnp.float32),
            pltpu.VMEM((chunk,), jnp.int32),
            pltpu.VMEM((chunk,), jnp.float32),
            pltpu.VMEM_SHARED((max(PASS_ROWS) * N,), jnp.float32),
            pltpu.SemaphoreType.DMA,
        ],
    )
    return grid_kernel(rows, cols, vals, zeros).reshape(N, N)


_BN = 512


def _graph_layer_body(x_ref, d_ref, out_ref, *, add_x2):
    j = pl.program_id(0)
    xg = x_ref[...].astype(jnp.bfloat16)
    acc = lax.dot_general(xg, d_ref[...].astype(jnp.bfloat16),
                          (((1,), (0,)), ((), ())),
                          preferred_element_type=jnp.float32)
    out_ref[...] = acc + x_ref[:, pl.ds(j * _BN, _BN)]


def _graph_layer_body_res(x_ref, d_ref, x0_ref, out_ref):
    j = pl.program_id(0)
    xg = x_ref[...].astype(jnp.bfloat16)
    acc = lax.dot_general(xg, d_ref[...].astype(jnp.bfloat16),
                          (((1,), (0,)), ((), ())),
                          preferred_element_type=jnp.float32)
    out_ref[...] = acc + x_ref[:, pl.ds(j * _BN, _BN)] + x0_ref[...]


def _graph_layer(xg, d, x0=None):
    grid = N // _BN
    in_specs = [
        pl.BlockSpec((B, N), lambda j: (0, 0)),
        pl.BlockSpec((N, _BN), lambda j: (0, j)),
    ]
    args = [xg, d]
    if x0 is None:
        body = functools.partial(_graph_layer_body, add_x2=False)
    else:
        body = _graph_layer_body_res
        in_specs.append(pl.BlockSpec((B, _BN), lambda j: (0, j)))
        args.append(x0)
    return pl.pallas_call(
        body,
        grid=(grid,),
        in_specs=in_specs,
        out_specs=pl.BlockSpec((B, _BN), lambda j: (0, j)),
        out_shape=jax.ShapeDtypeStruct((B, N), jnp.float32),
    )(*args)


_BM = 128


def _mlp_body(xx_ref, ts_ref, wemb_ref, bemb_ref, w1_ref, b1_ref, w2_ref, b2_ref,
              out_ref):
    half = EMB // 2
    ts = ts_ref[0, 0, :]
    k = lax.broadcasted_iota(jnp.int32, (1, half), 1).astype(jnp.float32)
    freqs = jnp.exp(k * (-math.log(10000.0) / half))
    args = ts[:, None] * freqs
    te = jnp.concatenate([jnp.cos(args), jnp.sin(args)], axis=-1)
    emb = lax.dot_general(te, wemb_ref[...], (((1,), (1,)), ((), ())),
                          preferred_element_type=jnp.float32,
                          precision=lax.Precision.HIGHEST) + bemb_ref[...]
    xx = xx_ref[...]
    w1 = w1_ref[...].astype(jnp.bfloat16)
    pre1 = lax.dot_general(xx.astype(jnp.bfloat16), w1[:, :N],
                           (((1,), (1,)), ((), ())),
                           preferred_element_type=jnp.float32)
    pre1 = pre1 + lax.dot_general(emb.astype(jnp.bfloat16), w1[:, N:],
                                  (((1,), (1,)), ((), ())),
                                  preferred_element_type=jnp.float32)
    h1 = jnp.tanh(pre1 + b1_ref[...]).astype(jnp.bfloat16)
    pre2 = lax.dot_general(h1, w2_ref[...].astype(jnp.bfloat16),
                           (((1,), (1,)), ((), ())),
                           preferred_element_type=jnp.float32)
    out_ref[...] = jnp.tanh(pre2 + b2_ref[...]) + xx


def _mlp(xx, timesteps, W_emb, b_emb, W1, b1, W2, b2):
    grid = B // _BM
    ts3 = timesteps.astype(jnp.float32).reshape(grid, 1, _BM)
    return pl.pallas_call(
        _mlp_body,
        grid=(grid,),
        in_specs=[
            pl.BlockSpec((_BM, N), lambda i: (i, 0)),
            pl.BlockSpec((1, 1, _BM), lambda i: (i, 0, 0)),
            pl.BlockSpec((EMB, EMB), lambda i: (0, 0)),
            pl.BlockSpec((1, EMB), lambda i: (0, 0)),
            pl.BlockSpec((HID, N + EMB), lambda i: (0, 0)),
            pl.BlockSpec((1, HID), lambda i: (0, 0)),
            pl.BlockSpec((N, HID), lambda i: (0, 0)),
            pl.BlockSpec((1, N), lambda i: (0, 0)),
        ],
        out_specs=pl.BlockSpec((_BM, N), lambda i: (i, 0)),
        out_shape=jax.ShapeDtypeStruct((B, N), jnp.float32),
    )(xx, ts3, W_emb, b_emb.reshape(1, EMB), W1, b1.reshape(1, HID), W2,
      b2.reshape(1, N))


def kernel(x, timesteps, graph_rows, graph_cols, graph_vals,
           W_emb, b_emb, W1, b1, W2, b2):
    rows = graph_rows.astype(jnp.int32)
    cols = graph_cols.astype(jnp.int32)
    vals = graph_vals.astype(jnp.float32)
    d = _densify(rows, cols, vals)
    xg1 = _graph_layer(x, d)
    xx = _graph_layer(xg1, d, x0=x)
    return _mlp(xx, timesteps, W_emb, b_emb, W1, b1, W2, b2)


# speedup vs baseline: 13.2158x; 1.0341x over previous
import functools
import math

import jax
import jax.numpy as jnp
from jax import lax
from jax.experimental import pallas as pl
from jax.experimental.pallas import tpu as pltpu
from jax.experimental.pallas import tpu_sc as plsc

N = 4096
B = 1024
EMB = 128
HID = 1024

NC = 2
NS = 16
PASS_ROWS = (464, 464, 464, 464, 192)
NCHUNK = 8


def _densify_body(rows_hbm, cols_hbm, vals_hbm, zeros_hbm, d_hbm,
                  rows_v, cols_v, vals_v, idx_buf, val_buf, idx_buf2, val_buf2,
                  shared, sem):
    chunk = rows_v.shape[0]
    share = chunk * NCHUNK
    cid = lax.axis_index("c")
    sid = lax.axis_index("s")
    half = N // NC

    pass_base = 0
    for p, rows_p in enumerate(PASS_ROWS):
        base_c = cid * half + pass_base
        rpt = rows_p // NS

        pltpu.sync_copy(zeros_hbm.at[pl.ds(sid * rpt * N, rpt * N)],
                        shared.at[pl.ds(sid * rpt * N, rpt * N)])
        plsc.subcore_barrier()

        bufs = ((idx_buf, val_buf), (idx_buf2, val_buf2))
        descs = [None, None]
        for c in range(NCHUNK):
            ib, vb = bufs[c % 2]
            if descs[c % 2] is not None:
                descs[c % 2].wait()
            off = sid * share + c * chunk
            pltpu.sync_copy(rows_hbm.at[pl.ds(off, chunk)], rows_v)
            pltpu.sync_copy(cols_hbm.at[pl.ds(off, chunk)], cols_v)
            pltpu.sync_copy(vals_hbm.at[pl.ds(off, chunk)], vals_v)

            def scan_body(i, _):
                r16 = rows_v[pl.ds(i * 16, 16)]
                c16 = cols_v[pl.ds(i * 16, 16)]
                v16 = vals_v[pl.ds(i * 16, 16)]
                lc = c16 - base_c
                m = (lc >= 0) & (lc < rows_p)
                ib[pl.ds(i * 16, 16)] = jnp.where(m, lc * N + r16, -1)
                vb[pl.ds(i * 16, 16)] = jnp.where(m, v16, 0.0)
                return 0

            lax.fori_loop(0, chunk // 16, scan_body, 0)
            descs[c % 2] = pltpu.async_copy(
                vb, shared.at[plsc.Indices(ib, ignored_value=-1)], sem,
                add=True)
        for d in descs:
            if d is not None:
                d.wait()
        plsc.subcore_barrier()

        src = sid * rpt * N
        dst = (base_c + sid * rpt) * N
        pltpu.sync_copy(shared.at[pl.ds(src, rpt * N)],
                        d_hbm.at[pl.ds(dst, rpt * N)])
        plsc.subcore_barrier()
        pass_base += rows_p


def _densify(rows, cols, vals):
    nnz = rows.shape[0]
    chunk = -(-nnz // (NS * NCHUNK * 16)) * 16
    pad = NS * NCHUNK * chunk - nnz
    rows = jnp.pad(rows, (0, pad))
    cols = jnp.pad(cols, (0, pad), constant_values=-1)
    vals = jnp.pad(vals, (0, pad))
    zeros = jnp.zeros((max(PASS_ROWS) * N,), jnp.float32)

    grid_kernel = pl.kernel(
        _densify_body,
        out_type=jax.ShapeDtypeStruct((N * N,), jnp.float32),
        mesh=plsc.VectorSubcoreMesh(core_axis_name="c", subcore_axis_name="s"),
        scratch_types=[
            pltpu.VMEM((chunk,), jnp.int32),
            pltpu.VMEM((chunk,), jnp.int32),
            pltpu.VMEM((chunk,), jnp.float32),
            pltpu.VMEM((chunk,), jnp.int32),
            pltpu.VMEM((chunk,), jnp.float32),
            pltpu.VMEM((chunk,), jnp.int32),
            pltpu.VMEM((chunk,), jnp.float32),
            pltpu.VMEM_SHARED((max(PASS_ROWS) * N,), jnp.float32),
            pltpu.SemaphoreType.DMA,
        ],
    )
    return grid_kernel(rows, cols, vals, zeros).reshape(N, N)


_BN = 512


def _graph_layer_body(x_ref, d_ref, out_ref, *, add_x2):
    j = pl.program_id(0)
    xg = x_ref[...].astype(jnp.bfloat16)
    acc = lax.dot_general(xg, d_ref[...].astype(jnp.bfloat16),
                          (((1,), (0,)), ((), ())),
                          preferred_element_type=jnp.float32)
    out_ref[...] = acc + x_ref[:, pl.ds(j * _BN, _BN)]


def _graph_layer_body_res(x_ref, d_ref, x0_ref, out_ref):
    j = pl.program_id(0)
    xg = x_ref[...].astype(jnp.bfloat16)
    acc = lax.dot_general(xg, d_ref[...].astype(jnp.bfloat16),
                          (((1,), (0,)), ((), ())),
                          preferred_element_type=jnp.float32)
    out_ref[...] = acc + x_ref[:, pl.ds(j * _BN, _BN)] + x0_ref[...]


def _graph_layer(xg, d, x0=None):
    grid = N // _BN
    in_specs = [
        pl.BlockSpec((B, N), lambda j: (0, 0)),
        pl.BlockSpec((N, _BN), lambda j: (0, j)),
    ]
    args = [xg, d]
    if x0 is None:
        body = functools.partial(_graph_layer_body, add_x2=False)
    else:
        body = _graph_layer_body_res
        in_specs.append(pl.BlockSpec((B, _BN), lambda j: (0, j)))
        args.append(x0)
    return pl.pallas_call(
        body,
        grid=(grid,),
        in_specs=in_specs,
        out_specs=pl.BlockSpec((B, _BN), lambda j: (0, j)),
        out_shape=jax.ShapeDtypeStruct((B, N), jnp.float32),
    )(*args)


_BM = 128


def _mlp_body(xx_ref, ts_ref, wemb_ref, bemb_ref, w1_ref, b1_ref, w2_ref, b2_ref,
              out_ref):
    half = EMB // 2
    ts = ts_ref[0, 0, :]
    k = lax.broadcasted_iota(jnp.int32, (1, half), 1).astype(jnp.float32)
    freqs = jnp.exp(k * (-math.log(10000.0) / half))
    args = ts[:, None] * freqs
    te = jnp.concatenate([jnp.cos(args), jnp.sin(args)], axis=-1)
    emb = lax.dot_general(te, wemb_ref[...], (((1,), (1,)), ((), ())),
                          preferred_element_type=jnp.float32,
                          precision=lax.Precision.HIGHEST) + bemb_ref[...]
    xx = xx_ref[...]
    w1 = w1_ref[...].astype(jnp.bfloat16)
    pre1 = lax.dot_general(xx.astype(jnp.bfloat16), w1[:, :N],
                           (((1,), (1,)), ((), ())),
                           preferred_element_type=jnp.float32)
    pre1 = pre1 + lax.dot_general(emb.astype(jnp.bfloat16), w1[:, N:],
                                  (((1,), (1,)), ((), ())),
                                  preferred_element_type=jnp.float32)
    h1 = jnp.tanh(pre1 + b1_ref[...]).astype(jnp.bfloat16)
    pre2 = lax.dot_general(h1, w2_ref[...].astype(jnp.bfloat16),
                           (((1,), (1,)), ((), ())),
                           preferred_element_type=jnp.float32)
    out_ref[...] = jnp.tanh(pre2 + b2_ref[...]) + xx


def _mlp(xx, timesteps, W_emb, b_emb, W1, b1, W2, b2):
    grid = B // _BM
    ts3 = timesteps.astype(jnp.float32).reshape(grid, 1, _BM)
    return pl.pallas_call(
        _mlp_body,
        grid=(grid,),
        in_specs=[
            pl.BlockSpec((_BM, N), lambda i: (i, 0)),
            pl.BlockSpec((1, 1, _BM), lambda i: (i, 0, 0)),
            pl.BlockSpec((EMB, EMB), lambda i: (0, 0)),
            pl.BlockSpec((1, EMB), lambda i: (0, 0)),
            pl.BlockSpec((HID, N + EMB), lambda i: (0, 0)),
            pl.BlockSpec((1, HID), lambda i: (0, 0)),
            pl.BlockSpec((N, HID), lambda i: (0, 0)),
            pl.BlockSpec((1, N), lambda i: (0, 0)),
        ],
        out_specs=pl.BlockSpec((_BM, N), lambda i: (i, 0)),
        out_shape=jax.ShapeDtypeStruct((B, N), jnp.float32),
    )(xx, ts3, W_emb, b_emb.reshape(1, EMB), W1, b1.reshape(1, HID), W2,
      b2.reshape(1, N))


def kernel(x, timesteps, graph_rows, graph_cols, graph_vals,
           W_emb, b_emb, W1, b1, W2, b2):
    rows = graph_rows.astype(jnp.int32)
    cols = graph_cols.astype(jnp.int32)
    vals = graph_vals.astype(jnp.float32)
    d = _densify(rows, cols, vals)
    xg1 = _graph_layer(x, d)
    xx = _graph_layer(xg1, d, x0=x)
    return _mlp(xx, timesteps, W_emb, b_emb, W1, b1, W2, b2)


# speedup vs baseline: 13.4659x; 1.0189x over previous
import functools
import math

import jax
import jax.numpy as jnp
from jax import lax
from jax.experimental import pallas as pl
from jax.experimental.pallas import tpu as pltpu
from jax.experimental.pallas import tpu_sc as plsc

N = 4096
B = 1024
EMB = 128
HID = 1024

NC = 2
NS = 16
PASS_ROWS = (464, 464, 464, 464, 192)
NCHUNK = 8


def _densify_body(rows_hbm, cols_hbm, vals_hbm, zeros_hbm, d_hbm,
                  rows_v, cols_v, vals_v, idx_buf, val_buf, idx_buf2, val_buf2,
                  shared, sem):
    chunk = rows_v.shape[0]
    share = chunk * NCHUNK
    cid = lax.axis_index("c")
    sid = lax.axis_index("s")
    half = N // NC

    pass_base = 0
    for p, rows_p in enumerate(PASS_ROWS):
        base_c = cid * half + pass_base
        rpt = rows_p // NS

        pltpu.sync_copy(zeros_hbm.at[pl.ds(sid * rpt * N, rpt * N)],
                        shared.at[pl.ds(sid * rpt * N, rpt * N)])
        plsc.subcore_barrier()

        bufs = ((idx_buf, val_buf), (idx_buf2, val_buf2))
        descs = [None, None]
        for c in range(NCHUNK):
            ib, vb = bufs[c % 2]
            if descs[c % 2] is not None:
                descs[c % 2].wait()
            off = sid * share + c * chunk
            pltpu.sync_copy(rows_hbm.at[pl.ds(off, chunk)], rows_v)
            pltpu.sync_copy(cols_hbm.at[pl.ds(off, chunk)], cols_v)
            pltpu.sync_copy(vals_hbm.at[pl.ds(off, chunk)], vals_v)

            def scan_body(i, _):
                r16 = rows_v[pl.ds(i * 16, 16)]
                c16 = cols_v[pl.ds(i * 16, 16)]
                v16 = vals_v[pl.ds(i * 16, 16)]
                lc = c16 - base_c
                m = (lc >= 0) & (lc < rows_p)
                ib[pl.ds(i * 16, 16)] = jnp.where(m, lc * N + r16, -1)
                vb[pl.ds(i * 16, 16)] = jnp.where(m, v16, 0.0)
                return 0

            lax.fori_loop(0, chunk // 16, scan_body, 0)
            descs[c % 2] = pltpu.async_copy(
                vb, shared.at[plsc.Indices(ib, ignored_value=-1)], sem,
                add=True)
        for d in descs:
            if d is not None:
                d.wait()
        plsc.subcore_barrier()

        src = sid * rpt * N
        dst = (base_c + sid * rpt) * N
        pltpu.sync_copy(shared.at[pl.ds(src, rpt * N)],
                        d_hbm.at[pl.ds(dst, rpt * N)])
        plsc.subcore_barrier()
        pass_base += rows_p


def _densify(rows, cols, vals):
    nnz = rows.shape[0]
    chunk = -(-nnz // (NS * NCHUNK * 16)) * 16
    pad = NS * NCHUNK * chunk - nnz
    rows = jnp.pad(rows, (0, pad))
    cols = jnp.pad(cols, (0, pad), constant_values=-1)
    vals = jnp.pad(vals, (0, pad))
    zeros = jnp.zeros((max(PASS_ROWS) * N,), jnp.float32)

    grid_kernel = pl.kernel(
        _densify_body,
        out_type=jax.ShapeDtypeStruct((N * N,), jnp.float32),
        mesh=plsc.VectorSubcoreMesh(core_axis_name="c", subcore_axis_name="s"),
        scratch_types=[
            pltpu.VMEM((chunk,), jnp.int32),
            pltpu.VMEM((chunk,), jnp.int32),
            pltpu.VMEM((chunk,), jnp.float32),
            pltpu.VMEM((chunk,), jnp.int32),
            pltpu.VMEM((chunk,), jnp.float32),
            pltpu.VMEM((chunk,), jnp.int32),
            pltpu.VMEM((chunk,), jnp.float32),
            pltpu.VMEM_SHARED((max(PASS_ROWS) * N,), jnp.float32),
            pltpu.SemaphoreType.DMA,
        ],
    )
    return grid_kernel(rows, cols, vals, zeros).reshape(N, N)


_BN = 512


def _graph2_body(x_ref, d_ref, out_ref, xg1_ref):
    l = pl.program_id(0)
    j = pl.program_id(1)
    db = d_ref[...].astype(jnp.bfloat16)

    @pl.when(l == 0)
    def _phase0():
        acc = lax.dot_general(x_ref[...].astype(jnp.bfloat16), db,
                              (((1,), (0,)), ((), ())),
                              preferred_element_type=jnp.float32)
        xg1_ref[:, pl.ds(j * _BN, _BN)] = (
            acc + x_ref[:, pl.ds(j * _BN, _BN)]).astype(jnp.bfloat16)

    @pl.when(l == 1)
    def _phase1():
        acc = lax.dot_general(xg1_ref[...], db, (((1,), (0,)), ((), ())),
                              preferred_element_type=jnp.float32)
        out_ref[...] = (acc + xg1_ref[:, pl.ds(j * _BN, _BN)]
                        + x_ref[:, pl.ds(j * _BN, _BN)])


def _graph2(x, d):
    grid = N // _BN
    return pl.pallas_call(
        _graph2_body,
        grid=(2, grid),
        in_specs=[
            pl.BlockSpec((B, N), lambda l, j: (0, 0)),
            pl.BlockSpec((N, _BN), lambda l, j: (0, j)),
        ],
        out_specs=pl.BlockSpec((B, _BN), lambda l, j: (0, j)),
        out_shape=jax.ShapeDtypeStruct((B, N), jnp.float32),
        scratch_shapes=[pltpu.VMEM((B, N), jnp.bfloat16)],
    )(x, d)


_BM = 128


def _mlp_body(xx_ref, ts_ref, wemb_ref, bemb_ref, w1_ref, b1_ref, w2_ref, b2_ref,
              out_ref):
    half = EMB // 2
    ts = ts_ref[0, 0, :]
    k = lax.broadcasted_iota(jnp.int32, (1, half), 1).astype(jnp.float32)
    freqs = jnp.exp(k * (-math.log(10000.0) / half))
    args = ts[:, None] * freqs
    te = jnp.concatenate([jnp.cos(args), jnp.sin(args)], axis=-1)
    emb = lax.dot_general(te, wemb_ref[...], (((1,), (1,)), ((), ())),
                          preferred_element_type=jnp.float32,
                          precision=lax.Precision.HIGHEST) + bemb_ref[...]
    xx = xx_ref[...]
    w1 = w1_ref[...].astype(jnp.bfloat16)
    pre1 = lax.dot_general(xx.astype(jnp.bfloat16), w1[:, :N],
                           (((1,), (1,)), ((), ())),
                           preferred_element_type=jnp.float32)
    pre1 = pre1 + lax.dot_general(emb.astype(jnp.bfloat16), w1[:, N:],
                                  (((1,), (1,)), ((), ())),
                                  preferred_element_type=jnp.float32)
    h1 = jnp.tanh(pre1 + b1_ref[...]).astype(jnp.bfloat16)
    pre2 = lax.dot_general(h1, w2_ref[...].astype(jnp.bfloat16),
                           (((1,), (1,)), ((), ())),
                           preferred_element_type=jnp.float32)
    out_ref[...] = jnp.tanh(pre2 + b2_ref[...]) + xx


def _mlp(xx, timesteps, W_emb, b_emb, W1, b1, W2, b2):
    grid = B // _BM
    ts3 = timesteps.astype(jnp.float32).reshape(grid, 1, _BM)
    return pl.pallas_call(
        _mlp_body,
        grid=(grid,),
        in_specs=[
            pl.BlockSpec((_BM, N), lambda i: (i, 0)),
            pl.BlockSpec((1, 1, _BM), lambda i: (i, 0, 0)),
            pl.BlockSpec((EMB, EMB), lambda i: (0, 0)),
            pl.BlockSpec((1, EMB), lambda i: (0, 0)),
            pl.BlockSpec((HID, N + EMB), lambda i: (0, 0)),
            pl.BlockSpec((1, HID), lambda i: (0, 0)),
            pl.BlockSpec((N, HID), lambda i: (0, 0)),
            pl.BlockSpec((1, N), lambda i: (0, 0)),
        ],
        out_specs=pl.BlockSpec((_BM, N), lambda i: (i, 0)),
        out_shape=jax.ShapeDtypeStruct((B, N), jnp.float32),
    )(xx, ts3, W_emb, b_emb.reshape(1, EMB), W1, b1.reshape(1, HID), W2,
      b2.reshape(1, N))


def kernel(x, timesteps, graph_rows, graph_cols, graph_vals,
           W_emb, b_emb, W1, b1, W2, b2):
    rows = graph_rows.astype(jnp.int32)
    cols = graph_cols.astype(jnp.int32)
    vals = graph_vals.astype(jnp.float32)
    d = _densify(rows, cols, vals)
    xx = _graph2(x, d)
    return _mlp(xx, timesteps, W_emb, b_emb, W1, b1, W2, b2)


# speedup vs baseline: 16.6749x; 1.2383x over previous
import functools
import math

import jax
import jax.numpy as jnp
from jax import lax
from jax.experimental import pallas as pl
from jax.experimental.pallas import tpu as pltpu
from jax.experimental.pallas import tpu_sc as plsc

N = 4096
B = 1024
EMB = 128
HID = 1024

NC = 2
NS = 16
PASS_ROWS = (464, 464, 464, 464, 192)
NCHUNK = 8


def _densify_body(rows_hbm, cols_hbm, vals_hbm, zeros_hbm, d_hbm,
                  rows_v, cols_v, vals_v, idx_buf, val_buf, idx_buf2, val_buf2,
                  shared, sem, sem2):
    chunk = rows_v.shape[0]
    share = chunk * NCHUNK
    cid = lax.axis_index("c")
    sid = lax.axis_index("s")
    half = N // NC
    bufs = ((idx_buf, val_buf), (idx_buf2, val_buf2))

    def stage_scan(c, base_c, rows_p):
        ib, vb = bufs[c % 2]
        off = sid * share + c * chunk
        pltpu.sync_copy(rows_hbm.at[pl.ds(off, chunk)], rows_v)
        pltpu.sync_copy(cols_hbm.at[pl.ds(off, chunk)], cols_v)
        pltpu.sync_copy(vals_hbm.at[pl.ds(off, chunk)], vals_v)

        def scan_body(i, _):
            r16 = rows_v[pl.ds(i * 16, 16)]
            c16 = cols_v[pl.ds(i * 16, 16)]
            v16 = vals_v[pl.ds(i * 16, 16)]
            lc = c16 - base_c
            m = (lc >= 0) & (lc < rows_p)
            ib[pl.ds(i * 16, 16)] = jnp.where(m, lc * N + r16, -1)
            vb[pl.ds(i * 16, 16)] = jnp.where(m, v16, 0.0)
            return 0

        lax.fori_loop(0, chunk // 16, scan_body, 0)

    def scatter(c):
        ib, vb = bufs[c % 2]
        return pltpu.async_copy(
            vb, shared.at[plsc.Indices(ib, ignored_value=-1)], sem, add=True)

    wd = None
    pass_base = 0
    for p, rows_p in enumerate(PASS_ROWS):
        base_c = cid * half + pass_base
        rpt = rows_p // NS

        stage_scan(0, base_c, rows_p)
        if wd is not None:
            wd.wait()
        zd = pltpu.async_copy(zeros_hbm.at[pl.ds(sid * rpt * N, rpt * N)],
                              shared.at[pl.ds(sid * rpt * N, rpt * N)], sem2)
        stage_scan(1, base_c, rows_p)
        zd.wait()
        plsc.subcore_barrier()
        descs = [scatter(0), scatter(1)]
        for c in range(2, NCHUNK):
            descs[c % 2].wait()
            stage_scan(c, base_c, rows_p)
            descs[c % 2] = scatter(c)
        descs[0].wait()
        descs[1].wait()
        plsc.subcore_barrier()

        src = sid * rpt * N
        dst = (base_c + sid * rpt) * N
        wd = pltpu.async_copy(shared.at[pl.ds(src, rpt * N)],
                              d_hbm.at[pl.ds(dst, rpt * N)], sem2)
        pass_base += rows_p
    wd.wait()


def _densify(rows, cols, vals):
    nnz = rows.shape[0]
    chunk = -(-nnz // (NS * NCHUNK * 16)) * 16
    pad = NS * NCHUNK * chunk - nnz
    rows = jnp.pad(rows, (0, pad))
    cols = jnp.pad(cols, (0, pad), constant_values=-1)
    vals = jnp.pad(vals, (0, pad))
    zeros = jnp.zeros((max(PASS_ROWS) * N,), jnp.float32)

    grid_kernel = pl.kernel(
        _densify_body,
        out_type=jax.ShapeDtypeStruct((N * N,), jnp.float32),
        mesh=plsc.VectorSubcoreMesh(core_axis_name="c", subcore_axis_name="s"),
        scratch_types=[
            pltpu.VMEM((chunk,), jnp.int32),
            pltpu.VMEM((chunk,), jnp.int32),
            pltpu.VMEM((chunk,), jnp.float32),
            pltpu.VMEM((chunk,), jnp.int32),
            pltpu.VMEM((chunk,), jnp.float32),
            pltpu.VMEM((chunk,), jnp.int32),
            pltpu.VMEM((chunk,), jnp.float32),
            pltpu.VMEM_SHARED((max(PASS_ROWS) * N,), jnp.float32),
            pltpu.SemaphoreType.DMA,
            pltpu.SemaphoreType.DMA,
        ],
    )
    return grid_kernel(rows, cols, vals, zeros).reshape(N, N)


_BN = 512


def _graph2_body(x_ref, d_ref, out_ref, xg1_ref):
    l = pl.program_id(0)
    j = pl.program_id(1)
    db = d_ref[...].astype(jnp.bfloat16)

    @pl.when(l == 0)
    def _phase0():
        acc = lax.dot_general(x_ref[...].astype(jnp.bfloat16), db,
                              (((1,), (0,)), ((), ())),
                              preferred_element_type=jnp.float32)
        xg1_ref[:, pl.ds(j * _BN, _BN)] = (
            acc + x_ref[:, pl.ds(j * _BN, _BN)]).astype(jnp.bfloat16)

    @pl.when(l == 1)
    def _phase1():
        acc = lax.dot_general(xg1_ref[...], db, (((1,), (0,)), ((), ())),
                              preferred_element_type=jnp.float32)
        out_ref[...] = (acc + xg1_ref[:, pl.ds(j * _BN, _BN)]
                        + x_ref[:, pl.ds(j * _BN, _BN)])


def _graph2(x, d):
    grid = N // _BN
    return pl.pallas_call(
        _graph2_body,
        grid=(2, grid),
        in_specs=[
            pl.BlockSpec((B, N), lambda l, j: (0, 0)),
            pl.BlockSpec((N, _BN), lambda l, j: (0, j)),
        ],
        out_specs=pl.BlockSpec((B, _BN), lambda l, j: (0, j)),
        out_shape=jax.ShapeDtypeStruct((B, N), jnp.float32),
        scratch_shapes=[pltpu.VMEM((B, N), jnp.bfloat16)],
    )(x, d)


_BM = 256


def _mlp_body(xx_ref, ts_ref, wemb_ref, bemb_ref, w1_ref, b1_ref, w2_ref, b2_ref,
              out_ref):
    half = EMB // 2
    ts = ts_ref[0, 0, :]
    k = lax.broadcasted_iota(jnp.int32, (1, half), 1).astype(jnp.float32)
    freqs = jnp.exp(k * (-math.log(10000.0) / half))
    args = ts[:, None] * freqs
    te = jnp.concatenate([jnp.cos(args), jnp.sin(args)], axis=-1)
    emb = lax.dot_general(te, wemb_ref[...], (((1,), (1,)), ((), ())),
                          preferred_element_type=jnp.float32,
                          precision=lax.Precision.HIGHEST) + bemb_ref[...]
    xx = xx_ref[...]
    w1 = w1_ref[...].astype(jnp.bfloat16)
    pre1 = lax.dot_general(xx.astype(jnp.bfloat16), w1[:, :N],
                           (((1,), (1,)), ((), ())),
                           preferred_element_type=jnp.float32)
    pre1 = pre1 + lax.dot_general(emb.astype(jnp.bfloat16), w1[:, N:],
                                  (((1,), (1,)), ((), ())),
                                  preferred_element_type=jnp.float32)
    h1 = jnp.tanh(pre1 + b1_ref[...]).astype(jnp.bfloat16)
    pre2 = lax.dot_general(h1, w2_ref[...].astype(jnp.bfloat16),
                           (((1,), (1,)), ((), ())),
                           preferred_element_type=jnp.float32)
    out_ref[...] = jnp.tanh(pre2 + b2_ref[...]) + xx


def _mlp(xx, timesteps, W_emb, b_emb, W1, b1, W2, b2):
    grid = B // _BM
    ts3 = timesteps.astype(jnp.float32).reshape(grid, 1, _BM)
    return pl.pallas_call(
        _mlp_body,
        grid=(grid,),
        in_specs=[
            pl.BlockSpec((_BM, N), lambda i: (i, 0)),
            pl.BlockSpec((1, 1, _BM), lambda i: (i, 0, 0)),
            pl.BlockSpec((EMB, EMB), lambda i: (0, 0)),
            pl.BlockSpec((1, EMB), lambda i: (0, 0)),
            pl.BlockSpec((HID, N + EMB), lambda i: (0, 0)),
            pl.BlockSpec((1, HID), lambda i: (0, 0)),
            pl.BlockSpec((N, HID), lambda i: (0, 0)),
            pl.BlockSpec((1, N), lambda i: (0, 0)),
        ],
        out_specs=pl.BlockSpec((_BM, N), lambda i: (i, 0)),
        out_shape=jax.ShapeDtypeStruct((B, N), jnp.float32),
    )(xx, ts3, W_emb, b_emb.reshape(1, EMB), W1, b1.reshape(1, HID), W2,
      b2.reshape(1, N))


def kernel(x, timesteps, graph_rows, graph_cols, graph_vals,
           W_emb, b_emb, W1, b1, W2, b2):
    rows = graph_rows.astype(jnp.int32)
    cols = graph_cols.astype(jnp.int32)
    vals = graph_vals.astype(jnp.float32)
    d = _densify(rows, cols, vals)
    xx = _graph2(x, d)
    return _mlp(xx, timesteps, W_emb, b_emb, W1, b1, W2, b2)
